# baseline (speedup 1.0000x reference)
import jax
import jax.numpy as jnp
from jax import lax
from jax.experimental import pallas as pl
from jax.experimental.pallas import tpu as pltpu

INPUT_DIM = 256
NUM_RISK_BINS = 4
NUM_ENG_BINS = 4
NUM_TRANSIENT = 7
DROPOUT_STATE_ID = 24
NUM_BASE_STATES = NUM_RISK_BINS * NUM_ENG_BINS

BLK = 2048


def _disc_kernel(h_ref, dl_ref, w1_ref, b1_ref, rw_ref, rb_ref, ew_ref, eb_ref,
                 w2_ref, b2_ref, out_ref, w_scr):
    @pl.when(pl.program_id(0) == 0)
    def _assemble():
        w_scr[:, 0:64] = w1_ref[...]
        w_scr[:, 64:65] = rw_ref[...]
        w_scr[:, 65:66] = ew_ref[...]

    h = h_ref[...]
    st = lax.dot_general(w_scr[...], h, (((0,), (1,)), ((), ())),
                         preferred_element_type=jnp.float32)
    risk = jax.nn.sigmoid(st[64:65, :] + rb_ref[0, 0])
    eng = jax.nn.sigmoid(st[65:66, :] + eb_ref[0, 0])
    hidden = jnp.maximum(st[0:64, :] + b1_ref[...], 0.0)
    logits = lax.dot_general(w2_ref[...], hidden, (((0,), (0,)), ((), ())),
                             preferred_element_type=jnp.float32)
    logits = logits + b2_ref[...]
    tstate = jnp.argmax(logits, axis=0).astype(jnp.int32)
    rbin = jnp.clip((risk * NUM_RISK_BINS).astype(jnp.int32), 0, NUM_RISK_BINS - 1)
    ebin = jnp.clip((eng * NUM_ENG_BINS).astype(jnp.int32), 0, NUM_ENG_BINS - 1)
    base = (rbin * NUM_ENG_BINS + ebin)[0, :]
    final = jnp.where(risk[0, :] > 0.75, NUM_BASE_STATES + tstate, base)
    final = jnp.where(dl_ref[0, 0, :] == 1, jnp.int32(DROPOUT_STATE_ID), final)
    out_ref[0, 0, :] = final


def kernel(h, dropout_labels, risk_W, risk_b, eng_W, eng_b, t_W1, t_b1, t_W2, t_b2):
    B = h.shape[0]
    nblk = B // BLK

    dl = dropout_labels.astype(jnp.int32).reshape(nblk, 1, BLK)

    out = pl.pallas_call(
        _disc_kernel,
        grid=(nblk,),
        in_specs=[
            pl.BlockSpec((BLK, INPUT_DIM), lambda i: (i, 0)),
            pl.BlockSpec((1, 1, BLK), lambda i: (i, 0, 0)),
            pl.BlockSpec((INPUT_DIM, 64), lambda i: (0, 0)),
            pl.BlockSpec((64, 1), lambda i: (0, 0)),
            pl.BlockSpec((INPUT_DIM, 1), lambda i: (0, 0)),
            pl.BlockSpec((1, 1), lambda i: (0, 0)),
            pl.BlockSpec((INPUT_DIM, 1), lambda i: (0, 0)),
            pl.BlockSpec((1, 1), lambda i: (0, 0)),
            pl.BlockSpec((64, NUM_TRANSIENT), lambda i: (0, 0)),
            pl.BlockSpec((NUM_TRANSIENT, 1), lambda i: (0, 0)),
        ],
        out_specs=pl.BlockSpec((1, 1, BLK), lambda i: (i, 0, 0)),
        out_shape=jax.ShapeDtypeStruct((nblk, 1, BLK), jnp.int32),
        scratch_shapes=[pltpu.VMEM((INPUT_DIM, 66), jnp.float32)],
    )(h, dl, t_W1, t_b1.reshape(64, 1), risk_W, risk_b.reshape(1, 1),
      eng_W, eng_b.reshape(1, 1), t_W2, t_b2.reshape(NUM_TRANSIENT, 1))
    return out.reshape(B)


# speedup vs baseline: 1.3330x; 1.3330x over previous
import jax
import jax.numpy as jnp
from jax import lax
from jax.experimental import pallas as pl

INPUT_DIM = 256
NUM_RISK_BINS = 4
NUM_ENG_BINS = 4
NUM_TRANSIENT = 7
DROPOUT_STATE_ID = 24
NUM_BASE_STATES = NUM_RISK_BINS * NUM_ENG_BINS

BLK = 4096


def _disc_kernel(h_ref, dl_ref, wt_ref, bias_ref, w2t_ref, b2t_ref, out_ref):
    h = h_ref[...]
    st = lax.dot_general(wt_ref[...], h, (((1,), (1,)), ((), ())),
                         preferred_element_type=jnp.float32)
    st = st + bias_ref[...]
    risk = jax.nn.sigmoid(st[64:65, :])
    eng = jax.nn.sigmoid(st[65:66, :])
    hidden = jnp.maximum(st[0:64, :], 0.0)
    logits = jnp.dot(w2t_ref[...], hidden, preferred_element_type=jnp.float32)
    logits = logits + b2t_ref[...]
    tstate = jnp.argmax(logits, axis=0).astype(jnp.int32)
    rbin = jnp.clip((risk * NUM_RISK_BINS).astype(jnp.int32), 0, NUM_RISK_BINS - 1)
    ebin = jnp.clip((eng * NUM_ENG_BINS).astype(jnp.int32), 0, NUM_ENG_BINS - 1)
    base = (rbin * NUM_ENG_BINS + ebin)[0, :]
    final = jnp.where(risk[0, :] > 0.75, NUM_BASE_STATES + tstate, base)
    final = jnp.where(dl_ref[0, 0, :] == 1, jnp.int32(DROPOUT_STATE_ID), final)
    out_ref[0, 0, :] = final


def kernel(h, dropout_labels, risk_W, risk_b, eng_W, eng_b, t_W1, t_b1, t_W2, t_b2):
    B = h.shape[0]
    nblk = B // BLK

    wt = jnp.concatenate(
        [t_W1.T, risk_W.T, eng_W.T, jnp.zeros((62, INPUT_DIM), jnp.float32)], axis=0)
    bias = jnp.concatenate(
        [t_b1, risk_b, eng_b, jnp.zeros((62,), jnp.float32)]).reshape(128, 1)
    w2t = jnp.concatenate([t_W2.T, jnp.zeros((1, 64), jnp.float32)], axis=0)
    b2t = jnp.concatenate([t_b2, jnp.full((1,), -1e30, jnp.float32)]).reshape(8, 1)

    dl = dropout_labels.astype(jnp.int32).reshape(nblk, 1, BLK)

    out = pl.pallas_call(
        _disc_kernel,
        grid=(nblk,),
        in_specs=[
            pl.BlockSpec((BLK, INPUT_DIM), lambda i: (i, 0)),
            pl.BlockSpec((1, 1, BLK), lambda i: (i, 0, 0)),
            pl.BlockSpec((128, INPUT_DIM), lambda i: (0, 0)),
            pl.BlockSpec((128, 1), lambda i: (0, 0)),
            pl.BlockSpec((8, 64), lambda i: (0, 0)),
            pl.BlockSpec((8, 1), lambda i: (0, 0)),
        ],
        out_specs=pl.BlockSpec((1, 1, BLK), lambda i: (i, 0, 0)),
        out_shape=jax.ShapeDtypeStruct((nblk, 1, BLK), jnp.int32),
    )(h, dl, wt, bias, w2t, b2t)
    return out.reshape(B)


# speedup vs baseline: 1.3568x; 1.0178x over previous
import jax
import jax.numpy as jnp
from jax import lax
from jax.experimental import pallas as pl

INPUT_DIM = 256
NUM_RISK_BINS = 4
NUM_ENG_BINS = 4
NUM_TRANSIENT = 7
DROPOUT_STATE_ID = 24
NUM_BASE_STATES = NUM_RISK_BINS * NUM_ENG_BINS

BLK = 8192


def _disc_kernel(h_ref, dl_ref, wt_ref, bias_ref, w2t_ref, b2t_ref, out_ref):
    h = h_ref[...]
    st = lax.dot_general(wt_ref[...], h, (((1,), (1,)), ((), ())),
                         preferred_element_type=jnp.float32)
    st = st + bias_ref[...]
    risk = jax.nn.sigmoid(st[64:65, :])
    eng = jax.nn.sigmoid(st[65:66, :])
    hidden = jnp.maximum(st[0:64, :], 0.0)
    logits = jnp.dot(w2t_ref[...], hidden, preferred_element_type=jnp.float32)
    logits = logits + b2t_ref[...]
    tstate = jnp.argmax(logits, axis=0).astype(jnp.int32)
    rbin = jnp.clip((risk * NUM_RISK_BINS).astype(jnp.int32), 0, NUM_RISK_BINS - 1)
    ebin = jnp.clip((eng * NUM_ENG_BINS).astype(jnp.int32), 0, NUM_ENG_BINS - 1)
    base = (rbin * NUM_ENG_BINS + ebin)[0, :]
    final = jnp.where(risk[0, :] > 0.75, NUM_BASE_STATES + tstate, base)
    final = jnp.where(dl_ref[0, 0, :] == 1, jnp.int32(DROPOUT_STATE_ID), final)
    out_ref[0, 0, :] = final


def kernel(h, dropout_labels, risk_W, risk_b, eng_W, eng_b, t_W1, t_b1, t_W2, t_b2):
    B = h.shape[0]
    nblk = B // BLK

    wt = jnp.concatenate(
        [t_W1.T, risk_W.T, eng_W.T, jnp.zeros((62, INPUT_DIM), jnp.float32)], axis=0)
    bias = jnp.concatenate(
        [t_b1, risk_b, eng_b, jnp.zeros((62,), jnp.float32)]).reshape(128, 1)
    w2t = jnp.concatenate([t_W2.T, jnp.zeros((1, 64), jnp.float32)], axis=0)
    b2t = jnp.concatenate([t_b2, jnp.full((1,), -1e30, jnp.float32)]).reshape(8, 1)

    dl = dropout_labels.astype(jnp.int32).reshape(nblk, 1, BLK)

    out = pl.pallas_call(
        _disc_kernel,
        grid=(nblk,),
        in_specs=[
            pl.BlockSpec((BLK, INPUT_DIM), lambda i: (i, 0)),
            pl.BlockSpec((1, 1, BLK), lambda i: (i, 0, 0)),
            pl.BlockSpec((128, INPUT_DIM), lambda i: (0, 0)),
            pl.BlockSpec((128, 1), lambda i: (0, 0)),
            pl.BlockSpec((8, 64), lambda i: (0, 0)),
            pl.BlockSpec((8, 1), lambda i: (0, 0)),
        ],
        out_specs=pl.BlockSpec((1, 1, BLK), lambda i: (i, 0, 0)),
        out_shape=jax.ShapeDtypeStruct((nblk, 1, BLK), jnp.int32),
    )(h, dl, wt, bias, w2t, b2t)
    return out.reshape(B)
